# Initial kernel scaffold; baseline (speedup 1.0000x reference)
import jax
import jax.numpy as jnp
from jax.experimental import pallas as pl


def kernel(x, batch, Win0, bin0, Win1, bin1, Win2, bin2, We00, be00, We01, be01, We10, be10, We11, be11, Wo0, bo0, Wo1, bo1, Wo2, bo2):
    raise NotImplementedError("write your pallas kernel here")


# speedup vs baseline: 3.9820x; 3.9820x over previous
import jax
import jax.numpy as jnp
from jax.experimental import pallas as pl

N = 10000
NPG = 100
G = N // NPG
DIN = 128
H = 64
DOUT = 8
K = 16


def _elu(v):
    return jnp.where(v > 0, v, jnp.exp(jnp.minimum(v, 0.0)) - 1.0)


def _edge_conv(xg, Wa, Wb, b0, W1, b1):
    n2 = jnp.sum(xg * xg, axis=1, keepdims=True)
    gram = jax.lax.dot_general(xg, xg, (((1,), (1,)), ((), ())),
                               preferred_element_type=jnp.float32)
    d2 = n2 + n2.T - 2.0 * gram
    iota_r = jax.lax.broadcasted_iota(jnp.int32, (NPG, NPG), 0)
    iota_c = jax.lax.broadcasted_iota(jnp.int32, (NPG, NPG), 1)
    d2 = jnp.where(iota_r == iota_c, d2 + 1e9, d2)

    a = jnp.dot(xg, Wa, preferred_element_type=jnp.float32) + b0
    bm = jnp.dot(xg, Wb, preferred_element_type=jnp.float32)

    onehots = []
    for _ in range(K):
        m = jnp.min(d2, axis=1, keepdims=True)
        eq = d2 == m
        sel = jnp.min(jnp.where(eq, iota_c, NPG), axis=1, keepdims=True)
        oh = iota_c == sel
        d2 = jnp.where(oh, jnp.inf, d2)
        onehots.append(oh.astype(jnp.float32))
    ostack = jnp.concatenate(onehots, axis=0)
    bj = jnp.dot(ostack, bm, preferred_element_type=jnp.float32)
    at = jnp.concatenate([a] * K, axis=0)
    h1 = _elu(at + bj)
    h2 = _elu(jnp.dot(h1, W1, preferred_element_type=jnp.float32) + b1)
    acc = h2[0:NPG]
    for k in range(1, K):
        acc = jnp.maximum(acc, h2[k * NPG:(k + 1) * NPG])
    return acc


def _net_kernel(x_ref, win0_ref, bin0_ref, win1_ref, bin1_ref, win2_ref,
                bin2_ref, wa0_ref, wb0_ref, be00_ref, we01_ref, be01_ref,
                wa1_ref, wb1_ref, be10_ref, we11_ref, be11_ref, wo0_ref,
                bo0_ref, wo1_ref, bo1_ref, wo2_ref, bo2_ref, out_ref):
    xg = x_ref[0]
    h = _elu(jnp.dot(xg, win0_ref[...], preferred_element_type=jnp.float32)
             + bin0_ref[...])
    h = _elu(jnp.dot(h, win1_ref[...], preferred_element_type=jnp.float32)
             + bin1_ref[...])
    h = _elu(jnp.dot(h, win2_ref[...], preferred_element_type=jnp.float32)
             + bin2_ref[...])
    h = _edge_conv(h, wa0_ref[...], wb0_ref[...], be00_ref[...],
                   we01_ref[...], be01_ref[...])
    h = _edge_conv(h, wa1_ref[...], wb1_ref[...], be10_ref[...],
                   we11_ref[...], be11_ref[...])
    pooled = jnp.max(h, axis=0, keepdims=True)
    o = _elu(jnp.dot(pooled, wo0_ref[...], preferred_element_type=jnp.float32)
             + bo0_ref[...])
    o = _elu(jnp.dot(o, wo1_ref[...], preferred_element_type=jnp.float32)
             + bo1_ref[...])
    logits = (jnp.dot(o, wo2_ref[...], preferred_element_type=jnp.float32)
              + bo2_ref[...])
    m = jnp.max(logits, axis=1, keepdims=True)
    z = logits - m
    lse = jnp.log(jnp.sum(jnp.exp(z), axis=1, keepdims=True))
    out_ref[0] = z - lse


def _full(shape):
    return pl.BlockSpec(shape, lambda g: (0,) * len(shape))


def kernel(x, batch, Win0, bin0, Win1, bin1, Win2, bin2, We00, be00, We01,
           be01, We10, be10, We11, be11, Wo0, bo0, Wo1, bo1, Wo2, bo2):
    del batch
    xg = x.reshape(G, NPG, DIN)
    wa0 = We00[:H] - We00[H:]
    wb0 = We00[H:]
    wa1 = We10[:H] - We10[H:]
    wb1 = We10[H:]
    row = lambda v: v.reshape(1, -1)
    out = pl.pallas_call(
        _net_kernel,
        grid=(G,),
        in_specs=[
            pl.BlockSpec((1, NPG, DIN), lambda g: (g, 0, 0)),
            _full((DIN, H)), _full((1, H)),
            _full((H, H)), _full((1, H)),
            _full((H, H)), _full((1, H)),
            _full((H, H)), _full((H, H)), _full((1, H)),
            _full((H, H)), _full((1, H)),
            _full((H, H)), _full((H, H)), _full((1, H)),
            _full((H, H)), _full((1, H)),
            _full((H, H)), _full((1, H)),
            _full((H, H)), _full((1, H)),
            _full((H, DOUT)), _full((1, DOUT)),
        ],
        out_specs=pl.BlockSpec((1, 1, DOUT), lambda g: (g, 0, 0)),
        out_shape=jax.ShapeDtypeStruct((G, 1, DOUT), jnp.float32),
    )(xg, Win0, row(bin0), Win1, row(bin1), Win2, row(bin2),
      wa0, wb0, row(be00), We01, row(be01),
      wa1, wb1, row(be10), We11, row(be11),
      Wo0, row(bo0), Wo1, row(bo1), Wo2, row(bo2))
    return out.reshape(G, DOUT)


# speedup vs baseline: 5.4956x; 1.3801x over previous
import jax
import jax.numpy as jnp
from jax.experimental import pallas as pl
from jax.experimental.pallas import tpu as pltpu

N = 10000
NPG = 100
G = N // NPG
DIN = 128
H = 64
DOUT = 8
K = 16


def _elu(v):
    return jnp.where(v > 0, v, jnp.exp(v) - 1.0)


def _edge_conv(xg, Wa, Wb, b0, W1, b1):
    n2 = jnp.sum(xg * xg, axis=1, keepdims=True)
    gram = jax.lax.dot_general(xg, xg, (((1,), (1,)), ((), ())),
                               preferred_element_type=jnp.float32)
    d2 = n2 + n2.T - 2.0 * gram
    iota_r = jax.lax.broadcasted_iota(jnp.int32, (NPG, NPG), 0)
    iota_ci = jax.lax.broadcasted_iota(jnp.int32, (NPG, NPG), 1)
    iota_c = iota_ci.astype(jnp.float32)
    iota_r = iota_r.astype(jnp.float32)
    d2 = jnp.where(iota_r == iota_c, d2 + 1e9, d2)

    a = jnp.dot(xg, Wa, preferred_element_type=jnp.float32) + b0
    bm = jnp.dot(xg, Wb, preferred_element_type=jnp.float32)

    onehots = []
    for _ in range(K):
        m = jnp.min(d2, axis=1, keepdims=True)
        eq = d2 == m
        sel = jnp.min(jnp.where(eq, iota_c, jnp.float32(NPG)), axis=1,
                      keepdims=True)
        oh = iota_c == sel
        d2 = jnp.where(oh, jnp.inf, d2)
        onehots.append(oh.astype(jnp.float32))
    ostack = jnp.concatenate(onehots, axis=0)
    bj = jnp.dot(ostack, bm, preferred_element_type=jnp.float32)
    at = jnp.concatenate([a] * K, axis=0)
    h1 = _elu(at + bj)
    h2 = _elu(jnp.dot(h1, W1, preferred_element_type=jnp.float32) + b1)
    acc = h2[0:NPG]
    for k in range(1, K):
        acc = jnp.maximum(acc, h2[k * NPG:(k + 1) * NPG])
    return acc


def _net_kernel(x_ref, win0_ref, bin0_ref, win1_ref, bin1_ref, win2_ref,
                bin2_ref, wa0_ref, wb0_ref, be00_ref, we01_ref, be01_ref,
                wa1_ref, wb1_ref, be10_ref, we11_ref, be11_ref, wo0_ref,
                bo0_ref, wo1_ref, bo1_ref, wo2_ref, bo2_ref, out_ref):
    xg = x_ref[0]
    h = _elu(jnp.dot(xg, win0_ref[...], preferred_element_type=jnp.float32)
             + bin0_ref[...])
    h = _elu(jnp.dot(h, win1_ref[...], preferred_element_type=jnp.float32)
             + bin1_ref[...])
    h = _elu(jnp.dot(h, win2_ref[...], preferred_element_type=jnp.float32)
             + bin2_ref[...])
    h = _edge_conv(h, wa0_ref[...], wb0_ref[...], be00_ref[...],
                   we01_ref[...], be01_ref[...])
    h = _edge_conv(h, wa1_ref[...], wb1_ref[...], be10_ref[...],
                   we11_ref[...], be11_ref[...])
    pooled = jnp.max(h, axis=0, keepdims=True)
    o = _elu(jnp.dot(pooled, wo0_ref[...], preferred_element_type=jnp.float32)
             + bo0_ref[...])
    o = _elu(jnp.dot(o, wo1_ref[...], preferred_element_type=jnp.float32)
             + bo1_ref[...])
    logits = (jnp.dot(o, wo2_ref[...], preferred_element_type=jnp.float32)
              + bo2_ref[...])
    m = jnp.max(logits, axis=1, keepdims=True)
    z = logits - m
    lse = jnp.log(jnp.sum(jnp.exp(z), axis=1, keepdims=True))
    out_ref[0] = z - lse


def _full(shape):
    return pl.BlockSpec(shape, lambda g: (0,) * len(shape))


def kernel(x, batch, Win0, bin0, Win1, bin1, Win2, bin2, We00, be00, We01,
           be01, We10, be10, We11, be11, Wo0, bo0, Wo1, bo1, Wo2, bo2):
    del batch
    xg = x.reshape(G, NPG, DIN)
    wa0 = We00[:H] - We00[H:]
    wb0 = We00[H:]
    wa1 = We10[:H] - We10[H:]
    wb1 = We10[H:]
    row = lambda v: v.reshape(1, -1)
    out = pl.pallas_call(
        _net_kernel,
        grid=(G,),
        in_specs=[
            pl.BlockSpec((1, NPG, DIN), lambda g: (g, 0, 0)),
            _full((DIN, H)), _full((1, H)),
            _full((H, H)), _full((1, H)),
            _full((H, H)), _full((1, H)),
            _full((H, H)), _full((H, H)), _full((1, H)),
            _full((H, H)), _full((1, H)),
            _full((H, H)), _full((H, H)), _full((1, H)),
            _full((H, H)), _full((1, H)),
            _full((H, H)), _full((1, H)),
            _full((H, H)), _full((1, H)),
            _full((H, DOUT)), _full((1, DOUT)),
        ],
        out_specs=pl.BlockSpec((1, 1, DOUT), lambda g: (g, 0, 0)),
        out_shape=jax.ShapeDtypeStruct((G, 1, DOUT), jnp.float32),
        compiler_params=pltpu.CompilerParams(
            dimension_semantics=("parallel",)),
    )(xg, Win0, row(bin0), Win1, row(bin1), Win2, row(bin2),
      wa0, wb0, row(be00), We01, row(be01),
      wa1, wb1, row(be10), We11, row(be11),
      Wo0, row(bo0), Wo1, row(bo1), Wo2, row(bo2))
    return out.reshape(G, DOUT)


# speedup vs baseline: 14.0538x; 2.5573x over previous
import jax
import jax.numpy as jnp
from jax.experimental import pallas as pl
from jax.experimental.pallas import tpu as pltpu

N = 10000
NPG = 100
G = N // NPG
DIN = 128
H = 64
DOUT = 8
K = 16
GPB = 4
R = GPB * NPG


def _elu(v):
    return jnp.where(v > 0, v, jnp.exp(v) - 1.0)


def _edge_conv(xs, Wa, Wb, b0, W1, b1, diag, iota_c, ones_h):
    x2 = xs * xs
    scores = []
    for g in range(GPB):
        sub = xs[g * NPG:(g + 1) * NPG]
        gram = jax.lax.dot_general(sub, sub, (((1,), (1,)), ((), ())),
                                   preferred_element_type=jnp.float32)
        n2row = jax.lax.dot_general(ones_h, x2[g * NPG:(g + 1) * NPG],
                                    (((1,), (1,)), ((), ())),
                                    preferred_element_type=jnp.float32)
        scores.append(n2row - 2.0 * gram)
    d2 = jnp.concatenate(scores, axis=0)
    d2 = jnp.where(diag, d2 + 1e9, d2)

    a = jnp.dot(xs, Wa, preferred_element_type=jnp.float32) + b0
    bm = jnp.dot(xs, Wb, preferred_element_type=jnp.float32)

    onehots = []
    for _ in range(K):
        m = jnp.min(d2, axis=1, keepdims=True)
        eq = d2 == m
        sel = jnp.min(jnp.where(eq, iota_c, jnp.float32(NPG)), axis=1,
                      keepdims=True)
        oh = iota_c == sel
        d2 = jnp.where(oh, jnp.inf, d2)
        onehots.append(oh.astype(jnp.float32))
    bjs = []
    for g in range(GPB):
        ostack_g = jnp.concatenate(
            [o[g * NPG:(g + 1) * NPG] for o in onehots], axis=0)
        bjs.append(jnp.dot(ostack_g, bm[g * NPG:(g + 1) * NPG],
                           preferred_element_type=jnp.float32))
    bj = jnp.concatenate(bjs, axis=0)
    at = jnp.concatenate(
        [a[g * NPG:(g + 1) * NPG] for g in range(GPB) for _ in range(K)],
        axis=0)
    h1 = _elu(at + bj)
    h2 = _elu(jnp.dot(h1, W1, preferred_element_type=jnp.float32) + b1)
    accs = []
    for g in range(GPB):
        acc = h2[g * K * NPG:g * K * NPG + NPG]
        for k in range(1, K):
            base = g * K * NPG + k * NPG
            acc = jnp.maximum(acc, h2[base:base + NPG])
        accs.append(acc)
    return jnp.concatenate(accs, axis=0)


def _net_kernel(x_ref, win0_ref, bin0_ref, win1_ref, bin1_ref, win2_ref,
                bin2_ref, wa0_ref, wb0_ref, be00_ref, we01_ref, be01_ref,
                wa1_ref, wb1_ref, be10_ref, we11_ref, be11_ref, wo0_ref,
                bo0_ref, wo1_ref, bo1_ref, wo2_ref, bo2_ref, out_ref):
    xg = x_ref[0]
    h = _elu(jnp.dot(xg, win0_ref[...], preferred_element_type=jnp.float32)
             + bin0_ref[...])
    h = _elu(jnp.dot(h, win1_ref[...], preferred_element_type=jnp.float32)
             + bin1_ref[...])
    h = _elu(jnp.dot(h, win2_ref[...], preferred_element_type=jnp.float32)
             + bin2_ref[...])
    iota_c = jax.lax.broadcasted_iota(
        jnp.int32, (R, NPG), 1).astype(jnp.float32)
    rloc1 = jax.lax.broadcasted_iota(
        jnp.int32, (NPG, NPG), 0).astype(jnp.float32)
    row_local = jnp.concatenate([rloc1] * GPB, axis=0)
    diag = row_local == iota_c
    ones_h = jnp.ones((1, H), jnp.float32)
    h = _edge_conv(h, wa0_ref[...], wb0_ref[...], be00_ref[...],
                   we01_ref[...], be01_ref[...], diag, iota_c, ones_h)
    h = _edge_conv(h, wa1_ref[...], wb1_ref[...], be10_ref[...],
                   we11_ref[...], be11_ref[...], diag, iota_c, ones_h)
    pooled = jnp.concatenate(
        [jnp.max(h[g * NPG:(g + 1) * NPG], axis=0, keepdims=True)
         for g in range(GPB)], axis=0)
    o = _elu(jnp.dot(pooled, wo0_ref[...], preferred_element_type=jnp.float32)
             + bo0_ref[...])
    o = _elu(jnp.dot(o, wo1_ref[...], preferred_element_type=jnp.float32)
             + bo1_ref[...])
    logits = (jnp.dot(o, wo2_ref[...], preferred_element_type=jnp.float32)
              + bo2_ref[...])
    m = jnp.max(logits, axis=1, keepdims=True)
    z = logits - m
    lse = jnp.log(jnp.sum(jnp.exp(z), axis=1, keepdims=True))
    out_ref[0] = z - lse


def _full(shape):
    return pl.BlockSpec(shape, lambda g: (0,) * len(shape))


def kernel(x, batch, Win0, bin0, Win1, bin1, Win2, bin2, We00, be00, We01,
           be01, We10, be10, We11, be11, Wo0, bo0, Wo1, bo1, Wo2, bo2):
    del batch
    xg = x.reshape(G // GPB, R, DIN)
    wa0 = We00[:H] - We00[H:]
    wb0 = We00[H:]
    wa1 = We10[:H] - We10[H:]
    wb1 = We10[H:]
    row = lambda v: v.reshape(1, -1)
    out = pl.pallas_call(
        _net_kernel,
        grid=(G // GPB,),
        in_specs=[
            pl.BlockSpec((1, R, DIN), lambda g: (g, 0, 0)),
            _full((DIN, H)), _full((1, H)),
            _full((H, H)), _full((1, H)),
            _full((H, H)), _full((1, H)),
            _full((H, H)), _full((H, H)), _full((1, H)),
            _full((H, H)), _full((1, H)),
            _full((H, H)), _full((H, H)), _full((1, H)),
            _full((H, H)), _full((1, H)),
            _full((H, H)), _full((1, H)),
            _full((H, H)), _full((1, H)),
            _full((H, DOUT)), _full((1, DOUT)),
        ],
        out_specs=pl.BlockSpec((1, GPB, DOUT), lambda g: (g, 0, 0)),
        out_shape=jax.ShapeDtypeStruct((G // GPB, GPB, DOUT), jnp.float32),
        compiler_params=pltpu.CompilerParams(
            dimension_semantics=("parallel",)),
    )(xg, Win0, row(bin0), Win1, row(bin1), Win2, row(bin2),
      wa0, wb0, row(be00), We01, row(be01),
      wa1, wb1, row(be10), We11, row(be11),
      Wo0, row(bo0), Wo1, row(bo1), Wo2, row(bo2))
    return out.reshape(G, DOUT)
